# Initial kernel scaffold; baseline (speedup 1.0000x reference)
import jax
import jax.numpy as jnp
from jax.experimental import pallas as pl


def kernel(feat_src, h_self, edge_index):
    raise NotImplementedError("write your pallas kernel here")


# speedup vs baseline: 7.7447x; 7.7447x over previous
import functools

import jax
import jax.numpy as jnp
from jax import lax
from jax.experimental import pallas as pl
from jax.experimental.pallas import tpu as pltpu
from jax.experimental.pallas import tpu_sc as plsc

_N = 10000
_E = 320000
_D = 128
_CHUNK = 128
_NCHUNKS = _E // _CHUNK
_NP = 10240
_NC = 2
_NS = 16

_f32 = jnp.float32


def _sc_partials(feat_src, edge_index):
  mesh = plsc.VectorSubcoreMesh(core_axis_name="c", subcore_axis_name="s")
  chunks_per_core = _NCHUNKS // _NC
  rows_per_tile = _NP // _NS
  deg_per_tile = _NP // _NS

  @functools.partial(
      pl.kernel,
      mesh=mesh,
      out_type=(
          jax.ShapeDtypeStruct((_NC, _NP, _D), _f32),
          jax.ShapeDtypeStruct((_NC, _NP), _f32),
      ),
      scratch_types=[
          pltpu.VMEM((_CHUNK, _D), _f32),
          pltpu.VMEM((_CHUNK,), jnp.int32),
          pltpu.VMEM((1, _CHUNK), jnp.int32),
          pltpu.VMEM((_CHUNK,), _f32),
          pltpu.VMEM((deg_per_tile,), _f32),
          pltpu.VMEM_SHARED((_NP, _D), _f32),
          pltpu.VMEM_SHARED((_NP,), _f32),
          pltpu.SemaphoreType.DMA,
      ],
  )
  def body(feat_hbm, edge_hbm, zeros_hbm, psum_out, pdeg_out,
           rbuf, sidx, didx, ones, zdeg, acc, deg, sem):
    c = lax.axis_index("c")
    s = lax.axis_index("s")

    for i in range(_CHUNK // 16):
      ones[pl.ds(16 * i, 16)] = jnp.full((16,), 1.0, _f32)
    for i in range(deg_per_tile // 16):
      zdeg[pl.ds(16 * i, 16)] = jnp.zeros((16,), _f32)

    base = s * rows_per_tile
    pltpu.sync_copy(zeros_hbm.at[pl.ds(base, rows_per_tile), :],
                    acc.at[pl.ds(base, rows_per_tile), :])
    pltpu.sync_copy(zdeg, deg.at[pl.ds(s * deg_per_tile, deg_per_tile)])
    plsc.subcore_barrier()

    lo = c * chunks_per_core + (s * chunks_per_core) // _NS
    hi = c * chunks_per_core + ((s + 1) * chunks_per_core) // _NS

    def step(j, carry):
      off = j * _CHUNK
      pltpu.sync_copy(edge_hbm.at[0, pl.ds(off, _CHUNK)], sidx)
      pltpu.sync_copy(edge_hbm.at[1, pl.ds(off, _CHUNK)], didx.at[0])
      pltpu.async_copy(feat_hbm.at[sidx], rbuf, sem).wait()
      pltpu.sync_copy(rbuf, acc.at[didx.at[0]], add=True)
      pltpu.sync_copy(ones, deg.at[didx.at[0]], add=True)
      return carry

    lax.fori_loop(lo, hi, step, 0)
    plsc.subcore_barrier()

    pltpu.sync_copy(acc.at[pl.ds(base, rows_per_tile), :],
                    psum_out.at[c, pl.ds(base, rows_per_tile), :])
    pltpu.sync_copy(deg.at[pl.ds(s * deg_per_tile, deg_per_tile)],
                    pdeg_out.at[c, pl.ds(s * deg_per_tile, deg_per_tile)])

  return body(feat_src, edge_index, jnp.zeros((_NP, _D), _f32))


def _combine(psum, pdeg, h_self):
  p0, p1 = psum[0], psum[1]
  d0 = pdeg[0].reshape(-1, 1)
  d1 = pdeg[1].reshape(-1, 1)
  rows = 1000
  grid = (_N // rows,)

  def body(p0_ref, p1_ref, d0_ref, d1_ref, h_ref, o_ref):
    degree = jnp.maximum(d0_ref[...] + d1_ref[...], 1.0)
    o_ref[...] = jnp.maximum(
        (p0_ref[...] + p1_ref[...]) / degree + h_ref[...], 0.0)

  return pl.pallas_call(
      body,
      grid=grid,
      in_specs=[
          pl.BlockSpec((rows, _D), lambda i: (i, 0)),
          pl.BlockSpec((rows, _D), lambda i: (i, 0)),
          pl.BlockSpec((rows, 1), lambda i: (i, 0)),
          pl.BlockSpec((rows, 1), lambda i: (i, 0)),
          pl.BlockSpec((rows, _D), lambda i: (i, 0)),
      ],
      out_specs=pl.BlockSpec((rows, _D), lambda i: (i, 0)),
      out_shape=jax.ShapeDtypeStruct((_N, _D), _f32),
  )(p0, p1, d0, d1, h_self)


def kernel(feat_src, h_self, edge_index):
  psum, pdeg = _sc_partials(feat_src, edge_index)
  return _combine(psum, pdeg, h_self)
